# baseline (speedup 1.0000x reference)
import jax
import jax.numpy as jnp
from jax import lax
from jax.experimental import pallas as pl
from jax.experimental.pallas import tpu as pltpu
from jax.experimental.pallas import tpu_sc as plsc

NC = 2
NS = 16
LANES = 16
NW = NC * NS


def _sc_segment_matvec(xT, src, dst, w, zeros_blk):
    D, N = xT.shape
    E = src.shape[0]
    F = D // NW
    CHUNK = 8000
    NCHUNK = E // CHUNK
    GROUPS = CHUNK // LANES
    assert E % CHUNK == 0 and CHUNK % LANES == 0 and D % NW == 0

    mesh = plsc.VectorSubcoreMesh(core_axis_name="c", subcore_axis_name="s")

    def body(xT_hbm, src_hbm, dst_hbm, w_hbm, zero_hbm, agg_hbm,
             xblk, aggblk, srcb0, srcb1, dstb0, dstb1, wb0, wb1, sem0, sem1):
        srcb = (srcb0, srcb1)
        dstb = (dstb0, dstb1)
        wb = (wb0, wb1)
        c = lax.axis_index("c")
        s = lax.axis_index("s")
        wid = s * NC + c
        ebase = wid * (F * N)
        pltpu.sync_copy(xT_hbm.at[pl.ds(ebase, F * N)], xblk)
        pltpu.sync_copy(zero_hbm, aggblk)

        sems = (sem0, sem1)

        def start(ci, slot):
            base = ci * CHUNK
            return (
                pltpu.async_copy(src_hbm.at[pl.ds(base, CHUNK)], srcb[slot], sems[slot]),
                pltpu.async_copy(dst_hbm.at[pl.ds(base, CHUNK)], dstb[slot], sems[slot]),
                pltpu.async_copy(w_hbm.at[pl.ds(base, CHUNK)], wb[slot], sems[slot]),
            )

        for s01 in range(2):
            start(s01, s01)

        def pair(cp, carry):
            for s01 in range(2):
                ci = 2 * cp + s01
                pltpu.make_async_copy(src_hbm.at[pl.ds(0, CHUNK)], srcb[s01], sems[s01]).wait()
                pltpu.make_async_copy(dst_hbm.at[pl.ds(0, CHUNK)], dstb[s01], sems[s01]).wait()
                pltpu.make_async_copy(w_hbm.at[pl.ds(0, CHUNK)], wb[s01], sems[s01]).wait()

                def _grp(gi, c2):
                    off = gi * LANES
                    sv = srcb[s01][pl.ds(off, LANES)]
                    dv = dstb[s01][pl.ds(off, LANES)]
                    wv = wb[s01][pl.ds(off, LANES)]
                    for f in range(F):
                        fofs = jnp.full((LANES,), f * N, dtype=jnp.int32)
                        g = plsc.load_gather(xblk, [sv + fofs])
                        plsc.addupdate_scatter(aggblk, [dv + fofs], g * wv)
                    return c2

                lax.fori_loop(0, GROUPS, _grp, 0, unroll=10)

                @pl.when(ci + 2 < NCHUNK)
                def _():
                    base = (ci + 2) * CHUNK
                    pltpu.make_async_copy(src_hbm.at[pl.ds(base, CHUNK)], srcb[s01], sems[s01]).start()
                    pltpu.make_async_copy(dst_hbm.at[pl.ds(base, CHUNK)], dstb[s01], sems[s01]).start()
                    pltpu.make_async_copy(w_hbm.at[pl.ds(base, CHUNK)], wb[s01], sems[s01]).start()
            return carry

        lax.fori_loop(0, NCHUNK // 2, pair, 0)

        pltpu.sync_copy(aggblk, agg_hbm.at[pl.ds(ebase, F * N)])

    run = pl.kernel(
        body,
        out_type=jax.ShapeDtypeStruct((D * N,), jnp.float32),
        mesh=mesh,
        compiler_params=pltpu.CompilerParams(needs_layout_passes=False),
        scratch_types=[
            pltpu.VMEM((F * N,), jnp.float32),
            pltpu.VMEM((F * N,), jnp.float32),
            pltpu.VMEM((CHUNK,), jnp.int32),
            pltpu.VMEM((CHUNK,), jnp.int32),
            pltpu.VMEM((CHUNK,), jnp.int32),
            pltpu.VMEM((CHUNK,), jnp.int32),
            pltpu.VMEM((CHUNK,), jnp.float32),
            pltpu.VMEM((CHUNK,), jnp.float32),
            pltpu.SemaphoreType.DMA,
            pltpu.SemaphoreType.DMA,
        ],
    )
    return run(xT.reshape(D * N), src, dst, w, zeros_blk.reshape(-1)).reshape(D, N)


def _tc_edge_weights(wvec, bvec, c0, c1, c2):
    R = c0.shape[0]

    def body(w_ref, b_ref, c0_ref, c1_ref, c2_ref, o_ref):
        o_ref[...] = jnp.maximum(
            c0_ref[...] * w_ref[0] + c1_ref[...] * w_ref[1]
            + c2_ref[...] * w_ref[2] + b_ref[0], 0.0)

    return pl.pallas_call(
        body,
        out_shape=jax.ShapeDtypeStruct((R, 128), jnp.float32),
        in_specs=[pl.BlockSpec(memory_space=pltpu.SMEM),
                  pl.BlockSpec(memory_space=pltpu.SMEM),
                  pl.BlockSpec((R, 128), lambda: (0, 0)),
                  pl.BlockSpec((R, 128), lambda: (0, 0)),
                  pl.BlockSpec((R, 128), lambda: (0, 0))],
        out_specs=pl.BlockSpec((R, 128), lambda: (0, 0)),
    )(wvec, bvec, c0, c1, c2)


def _tc_fuse_transpose(a, b, c):
    N, D = a.shape

    def body(a_ref, b_ref, c_ref, o_ref):
        o_ref[...] = (a_ref[...] + b_ref[...] + c_ref[...]).T

    return pl.pallas_call(
        body,
        out_shape=jax.ShapeDtypeStruct((D, N), jnp.float32),
        in_specs=[pl.BlockSpec((N, D), lambda: (0, 0))] * 3,
        out_specs=pl.BlockSpec((D, N), lambda: (0, 0)),
    )(a, b, c)


def _tc_layer(aggT, xT, W, b2d):
    D, N = aggT.shape

    def body(W_ref, b_ref, agg_ref, x_ref, o_ref):
        h = lax.dot_general(W_ref[...], agg_ref[...], (((0,), (0,)), ((), ())),
                            precision=lax.Precision.HIGHEST,
                            preferred_element_type=jnp.float32)
        o_ref[...] = jnp.maximum(h + b_ref[...], 0.0) + x_ref[...]

    return pl.pallas_call(
        body,
        out_shape=jax.ShapeDtypeStruct((D, N), jnp.float32),
        in_specs=[pl.BlockSpec((D, D), lambda: (0, 0)),
                  pl.BlockSpec((D, 1), lambda: (0, 0)),
                  pl.BlockSpec((D, N), lambda: (0, 0)),
                  pl.BlockSpec((D, N), lambda: (0, 0))],
        out_specs=pl.BlockSpec((D, N), lambda: (0, 0)),
    )(W, b2d, aggT, xT)


def _tc_final(xT, fcWp, fcbp):
    D, N = xT.shape
    P = fcWp.shape[1]

    def body(w_ref, b_ref, x_ref, o_ref):
        o_ref[...] = lax.dot_general(w_ref[...], x_ref[...], (((0,), (0,)), ((), ())),
                                     precision=lax.Precision.HIGHEST,
                                     preferred_element_type=jnp.float32) + b_ref[...]

    return pl.pallas_call(
        body,
        out_shape=jax.ShapeDtypeStruct((P, N), jnp.float32),
        in_specs=[pl.BlockSpec((D, P), lambda: (0, 0)),
                  pl.BlockSpec((P, 1), lambda: (0, 0)),
                  pl.BlockSpec((D, N), lambda: (0, 0))],
        out_specs=pl.BlockSpec((P, N), lambda: (0, 0)),
    )(fcWp, fcbp, xT)


def kernel(x_struct, x_seq, edgeIndex, edgeAttribute, x_antiberty, token_seq, node_size,
           attr_W, attr_b, W0, b0, W1, b1, W2, b2, W3, b3, fc_W, fc_b):
    N, D = x_struct.shape
    E = edgeIndex.shape[1]
    OUT = fc_W.shape[1]

    src = edgeIndex[0]
    dst = edgeIndex[1]
    R = E // 128
    c0 = edgeAttribute[:, 0].reshape(R, 128)
    c1 = edgeAttribute[:, 1].reshape(R, 128)
    c2 = edgeAttribute[:, 2].reshape(R, 128)

    atb = _tc_edge_weights(attr_W.ravel(), attr_b, c0, c1, c2).ravel()
    xT = _tc_fuse_transpose(x_struct, x_seq, x_antiberty)

    zeros_blk = jnp.zeros((D // NW, N), jnp.float32)
    for W, b in ((W0, b0), (W1, b1), (W2, b2), (W3, b3)):
        aggT = _sc_segment_matvec(xT, src, dst, atb, zeros_blk)
        xT = _tc_layer(aggT, xT, W, b.reshape(D, 1))

    P = 8
    fcWp = jnp.zeros((D, P), fc_W.dtype).at[:, :OUT].set(fc_W)
    fcbp = jnp.zeros((P, 1), fc_b.dtype).at[:OUT, 0].set(fc_b)
    outp = _tc_final(xT, fcWp, fcbp)
    return outp[:OUT, :].T


# speedup vs baseline: 1.3407x; 1.3407x over previous
import jax
import jax.numpy as jnp
from jax import lax
from jax.experimental import pallas as pl
from jax.experimental.pallas import tpu as pltpu
from jax.experimental.pallas import tpu_sc as plsc

NC = 2
NS = 16
LANES = 16
NW = NC * NS

CH = 64
PT = 10240
NCH = PT // CH
NPAD = 10240


def _sc_segment_rows(x, pk, wp, zeros_nd):
    N, D = x.shape

    mesh = plsc.VectorSubcoreMesh(core_axis_name="c", subcore_axis_name="s")

    def body(x_hbm, pk_hbm, w_hbm, zero_hbm, out_hbm,
             pk0, pk1, sidx0, sidx1, di0, di1, di2, di3, wb0, wb1, wb2, wb3,
             in0, in1, ou0, ou1, aggS,
             spk0, spk1, sw0, sw1, sw2, sw3, sg0, sg1, ss0, ss1):
        c = lax.axis_index("c")
        s = lax.axis_index("s")
        wid = c * NS + s
        ebase = wid * PT
        pkb = (pk0, pk1)
        sidx = (sidx0, sidx1)
        didx = (di0, di1, di2, di3)
        wbs = (wb0, wb1, wb2, wb3)
        inb = (in0, in1)
        oub = (ou0, ou1)
        spk = (spk0, spk1)
        sw = (sw0, sw1, sw2, sw3)
        sg = (sg0, sg1)
        ss = (ss0, ss1)

        def unpack(pslot, islot, dslot):
            for g in range(CH // LANES):
                pv = pkb[pslot][pl.ds(g * LANES, LANES)]
                sidx[islot][pl.ds(g * LANES, LANES)] = lax.shift_right_logical(pv, 14)
                didx[dslot][pl.ds(g * LANES, LANES)] = lax.bitwise_and(pv, 16383)

        rows = NPAD // NS
        rbase = s * rows
        pltpu.sync_copy(zero_hbm.at[pl.ds(rbase, rows)], aggS.at[pl.ds(rbase, rows)])
        plsc.subcore_barrier()

        for k in (0, 1):
            base = ebase + k * CH
            pltpu.async_copy(pk_hbm.at[pl.ds(base, CH)], pkb[k], spk[k])
            pltpu.async_copy(w_hbm.at[pl.ds(base, CH)], wbs[k], sw[k])
        pltpu.make_async_copy(pk_hbm.at[pl.ds(0, CH)], pkb[0], spk[0]).wait()
        unpack(0, 0, 0)
        pltpu.async_copy(x_hbm.at[sidx[0]], inb[0], sg[0])

        def quad(kq, carry):
            for par in range(4):
                k = 4 * kq + par
                cur = par % 2
                nxt = (par + 1) % 2

                @pl.when(k + 1 < NCH)
                def _():
                    pltpu.make_async_copy(pk_hbm.at[pl.ds(0, CH)], pkb[nxt], spk[nxt]).wait()
                    unpack(nxt, nxt, (par + 1) % 4)
                    pltpu.async_copy(x_hbm.at[sidx[nxt]], inb[nxt], sg[nxt])

                @pl.when(k + 2 < NCH)
                def _():
                    base = ebase + (k + 2) * CH
                    pltpu.async_copy(pk_hbm.at[pl.ds(base, CH)], pkb[cur], spk[cur])

                pltpu.make_async_copy(x_hbm.at[pl.ds(0, CH)], inb[cur], sg[cur]).wait()

                @pl.when(k >= 2)
                def _():
                    pltpu.make_async_copy(x_hbm.at[pl.ds(0, CH)], oub[cur], ss[cur]).wait()

                @pl.when(k + 2 < NCH)
                def _():
                    base = ebase + (k + 2) * CH
                    sl = (par + 2) % 4
                    pltpu.async_copy(w_hbm.at[pl.ds(base, CH)], wbs[sl], sw[sl])

                pltpu.make_async_copy(w_hbm.at[pl.ds(0, CH)], wbs[par], sw[par]).wait()

                def grp(g, c2):
                    gb = g * LANES
                    wv = wbs[par][pl.ds(gb, LANES)]
                    for e in range(LANES):
                        wsp = jnp.take_along_axis(
                            wv, jnp.full((LANES,), e, dtype=jnp.int32), axis=0)
                        row = gb + e
                        for fb in range(D // LANES):
                            seg = inb[cur][row, pl.ds(fb * LANES, LANES)]
                            oub[cur][row, pl.ds(fb * LANES, LANES)] = seg * wsp
                    return c2

                lax.fori_loop(0, CH // LANES, grp, 0)

                pltpu.async_copy(oub[cur], aggS.at[didx[par]], ss[cur], add=True)
            return carry

        lax.fori_loop(0, NCH // 4, quad, 0)

        for sl in range(2):
            pltpu.make_async_copy(x_hbm.at[pl.ds(0, CH)], oub[sl], ss[sl]).wait()
        plsc.subcore_barrier()
        pltpu.sync_copy(aggS.at[pl.ds(rbase, rows)], out_hbm.at[c].at[pl.ds(rbase, rows)])

    run = pl.kernel(
        body,
        out_type=jax.ShapeDtypeStruct((NC, NPAD, D), jnp.float32),
        mesh=mesh,
        compiler_params=pltpu.CompilerParams(needs_layout_passes=False),
        scratch_types=[
            pltpu.VMEM((CH,), jnp.int32),
            pltpu.VMEM((CH,), jnp.int32),
            pltpu.VMEM((CH,), jnp.int32),
            pltpu.VMEM((CH,), jnp.int32),
            pltpu.VMEM((CH,), jnp.int32),
            pltpu.VMEM((CH,), jnp.int32),
            pltpu.VMEM((CH,), jnp.int32),
            pltpu.VMEM((CH,), jnp.int32),
            pltpu.VMEM((CH,), jnp.float32),
            pltpu.VMEM((CH,), jnp.float32),
            pltpu.VMEM((CH,), jnp.float32),
            pltpu.VMEM((CH,), jnp.float32),
            pltpu.VMEM((CH, 128), jnp.float32),
            pltpu.VMEM((CH, 128), jnp.float32),
            pltpu.VMEM((CH, 128), jnp.float32),
            pltpu.VMEM((CH, 128), jnp.float32),
            pltpu.VMEM_SHARED((NPAD, 128), jnp.float32),
            pltpu.SemaphoreType.DMA,
            pltpu.SemaphoreType.DMA,
            pltpu.SemaphoreType.DMA,
            pltpu.SemaphoreType.DMA,
            pltpu.SemaphoreType.DMA,
            pltpu.SemaphoreType.DMA,
            pltpu.SemaphoreType.DMA,
            pltpu.SemaphoreType.DMA,
            pltpu.SemaphoreType.DMA,
            pltpu.SemaphoreType.DMA,
        ],
    )
    return run(x, pk, wp, zeros_nd)


def _tc_edge_weights(wvec, bvec, c0, c1, c2):
    R = c0.shape[0]

    def body(w_ref, b_ref, c0_ref, c1_ref, c2_ref, o_ref):
        o_ref[...] = jnp.maximum(
            c0_ref[...] * w_ref[0] + c1_ref[...] * w_ref[1]
            + c2_ref[...] * w_ref[2] + b_ref[0], 0.0)

    return pl.pallas_call(
        body,
        out_shape=jax.ShapeDtypeStruct((R, 128), jnp.float32),
        in_specs=[pl.BlockSpec(memory_space=pltpu.SMEM),
                  pl.BlockSpec(memory_space=pltpu.SMEM),
                  pl.BlockSpec((R, 128), lambda: (0, 0)),
                  pl.BlockSpec((R, 128), lambda: (0, 0)),
                  pl.BlockSpec((R, 128), lambda: (0, 0))],
        out_specs=pl.BlockSpec((R, 128), lambda: (0, 0)),
    )(wvec, bvec, c0, c1, c2)


def _tc_fuse(a, b, c):
    N, D = a.shape

    def body(a_ref, b_ref, c_ref, o_ref):
        o_ref[...] = a_ref[...] + b_ref[...] + c_ref[...]

    return pl.pallas_call(
        body,
        out_shape=jax.ShapeDtypeStruct((N, D), jnp.float32),
        in_specs=[pl.BlockSpec((N, D), lambda: (0, 0))] * 3,
        out_specs=pl.BlockSpec((N, D), lambda: (0, 0)),
    )(a, b, c)


def _tc_layer(parts, x, W, b2d):
    _, NP, D = parts.shape
    N = x.shape[0]

    def body(W_ref, b_ref, p_ref, x_ref, o_ref):
        agg = p_ref[0, :N, :] + p_ref[1, :N, :]
        h = lax.dot_general(agg, W_ref[...], (((1,), (0,)), ((), ())),
                            precision=lax.Precision.HIGHEST,
                            preferred_element_type=jnp.float32)
        o_ref[...] = jnp.maximum(h + b_ref[...], 0.0) + x_ref[...]

    return pl.pallas_call(
        body,
        out_shape=jax.ShapeDtypeStruct((N, D), jnp.float32),
        in_specs=[pl.BlockSpec((D, D), lambda: (0, 0)),
                  pl.BlockSpec((1, D), lambda: (0, 0)),
                  pl.BlockSpec((2, NP, D), lambda: (0, 0, 0)),
                  pl.BlockSpec((N, D), lambda: (0, 0))],
        out_specs=pl.BlockSpec((N, D), lambda: (0, 0)),
    )(W, b2d, parts, x)


def _tc_final(x, fcWp, fcbp):
    N, D = x.shape
    P = fcWp.shape[1]

    def body(w_ref, b_ref, x_ref, o_ref):
        o_ref[...] = lax.dot_general(x_ref[...], w_ref[...], (((1,), (0,)), ((), ())),
                                     precision=lax.Precision.HIGHEST,
                                     preferred_element_type=jnp.float32) + b_ref[...]

    return pl.pallas_call(
        body,
        out_shape=jax.ShapeDtypeStruct((N, P), jnp.float32),
        in_specs=[pl.BlockSpec((D, P), lambda: (0, 0)),
                  pl.BlockSpec((1, P), lambda: (0, 0)),
                  pl.BlockSpec((N, D), lambda: (0, 0))],
        out_specs=pl.BlockSpec((N, P), lambda: (0, 0)),
    )(fcWp, fcbp, x)


def kernel(x_struct, x_seq, edgeIndex, edgeAttribute, x_antiberty, token_seq, node_size,
           attr_W, attr_b, W0, b0, W1, b1, W2, b2, W3, b3, fc_W, fc_b):
    N, D = x_struct.shape
    E = edgeIndex.shape[1]
    OUT = fc_W.shape[1]

    src = edgeIndex[0]
    dst = edgeIndex[1]
    R = E // 128
    c0 = edgeAttribute[:, 0].reshape(R, 128)
    c1 = edgeAttribute[:, 1].reshape(R, 128)
    c2 = edgeAttribute[:, 2].reshape(R, 128)

    atb = _tc_edge_weights(attr_W.ravel(), attr_b, c0, c1, c2).ravel()

    EP = NW * PT
    padn = EP - E
    pk = jnp.left_shift(src, 14) | dst
    pkp = jnp.concatenate([pk, jnp.zeros((padn,), pk.dtype)])
    wp = jnp.concatenate([atb, jnp.zeros((padn,), atb.dtype)])

    x = _tc_fuse(x_struct, x_seq, x_antiberty)

    zeros_nd = jnp.zeros((NPAD, D), jnp.float32)
    for W, b in ((W0, b0), (W1, b1), (W2, b2), (W3, b3)):
        parts = _sc_segment_rows(x, pkp, wp, zeros_nd)
        x = _tc_layer(parts, x, W, b.reshape(1, D))

    P = 8
    fcWp = jnp.zeros((D, P), fc_W.dtype).at[:, :OUT].set(fc_W)
    fcbp = jnp.zeros((1, P), fc_b.dtype).at[0, :OUT].set(fc_b)
    outp = _tc_final(x, fcWp, fcbp)
    return outp[:, :OUT]
